# baseline (device time: 16271 ns/iter reference)
import jax
import jax.numpy as jnp
from jax import lax
from jax.experimental import pallas as pl
from jax.experimental.pallas import tpu as pltpu

N_DEV = 4
S = 2


def kernel(x, router_W, route_idx, expert_W):
    del router_W
    n, d = x.shape
    e_per, _, h = expert_W.shape
    hc = h // N_DEV
    nr = n // S

    def body(x_ref, idx_ref, w_ref, out_ref, stage_ref, rs_comm_ref,
             ag_stage_ref, ag_comm_ref, rs_send_sems, rs_recv_sems,
             ag_send_sems, ag_recv_sems):
        my_pos = lax.axis_index("i")

        barrier_sem = pltpu.get_barrier_semaphore()
        for o in range(1, N_DEV):
            pl.semaphore_signal(
                barrier_sem, inc=1,
                device_id=((my_pos + o) % N_DEV,),
                device_id_type=pl.DeviceIdType.MESH,
            )

        route = idx_ref[:, :]
        xv = x_ref[:, :]
        xm = [
            jnp.where(route == my_pos * e_per + e, xv, 0.0).astype(jnp.bfloat16)
            for e in range(e_per)
        ]

        def partial_chunk(col_start):
            chunk = jnp.zeros((n, hc), dtype=jnp.float32)
            for e in range(e_per):
                chunk = chunk + jnp.dot(
                    xm[e], w_ref[e, :, pl.ds(col_start, hc)].astype(jnp.bfloat16),
                    preferred_element_type=jnp.float32,
                )
            return chunk

        for j in range(N_DEV - 1):
            c = lax.rem(my_pos + 1 + j, N_DEV)
            stage_ref[j, :, :] = partial_chunk(c * hc).astype(jnp.bfloat16)
            if j == 0:
                pl.semaphore_wait(barrier_sem, N_DEV - 1)
            for s in range(S):
                rdma = pltpu.make_async_remote_copy(
                    src_ref=stage_ref.at[j, pl.ds(s * nr, nr), :],
                    dst_ref=rs_comm_ref.at[j, pl.ds(s * nr, nr), :],
                    send_sem=rs_send_sems.at[j, s],
                    recv_sem=rs_recv_sems.at[j, s],
                    device_id=(c,),
                    device_id_type=pl.DeviceIdType.MESH,
                )
                rdma.start()

        own = partial_chunk(my_pos * hc)

        for s in range(S):
            rows = pl.ds(s * nr, nr)
            for o in range(1, N_DEV):
                recv = pltpu.make_async_remote_copy(
                    src_ref=stage_ref.at[0, rows, :],
                    dst_ref=rs_comm_ref.at[o - 1, rows, :],
                    send_sem=rs_send_sems.at[o - 1, s],
                    recv_sem=rs_recv_sems.at[o - 1, s],
                    device_id=((my_pos + o) % N_DEV,),
                    device_id_type=pl.DeviceIdType.MESH,
                )
                recv.wait_recv()
            final = own[s * nr:(s + 1) * nr, :]
            for o in range(1, N_DEV):
                final = final + rs_comm_ref[o - 1, rows, :].astype(jnp.float32)
            finalb = final.astype(jnp.bfloat16)
            ag_stage_ref[rows, :] = finalb
            for o in range(1, N_DEV):
                rdma = pltpu.make_async_remote_copy(
                    src_ref=ag_stage_ref.at[rows, :],
                    dst_ref=ag_comm_ref.at[o - 1, rows, :],
                    send_sem=ag_send_sems.at[o - 1, s],
                    recv_sem=ag_recv_sems.at[o - 1, s],
                    device_id=((my_pos + o) % N_DEV,),
                    device_id_type=pl.DeviceIdType.MESH,
                )
                rdma.start()
            out_ref[rows, pl.ds(my_pos * hc, hc)] = finalb

        for s in range(S):
            rows = pl.ds(s * nr, nr)
            for o in range(1, N_DEV):
                recv = pltpu.make_async_remote_copy(
                    src_ref=ag_stage_ref.at[rows, :],
                    dst_ref=ag_comm_ref.at[o - 1, rows, :],
                    send_sem=ag_send_sems.at[o - 1, s],
                    recv_sem=ag_recv_sems.at[o - 1, s],
                    device_id=((my_pos + o) % N_DEV,),
                    device_id_type=pl.DeviceIdType.MESH,
                )
                recv.wait_recv()
                p = (my_pos - o) % N_DEV
                out_ref[rows, pl.ds(p * hc, hc)] = ag_comm_ref[o - 1, rows, :]

        for k in range(N_DEV - 1):
            for s in range(S):
                for sems in (rs_send_sems, ag_send_sems):
                    drain = pltpu.make_async_remote_copy(
                        src_ref=ag_stage_ref.at[pl.ds(s * nr, nr), :],
                        dst_ref=ag_comm_ref.at[k, pl.ds(s * nr, nr), :],
                        send_sem=sems.at[k, s],
                        recv_sem=ag_recv_sems.at[k, s],
                        device_id=(0,),
                        device_id_type=pl.DeviceIdType.MESH,
                    )
                    drain.wait_send()

    return pl.pallas_call(
        body,
        out_shape=jax.ShapeDtypeStruct((n, h), jnp.bfloat16),
        in_specs=[
            pl.BlockSpec(memory_space=pltpu.VMEM),
            pl.BlockSpec(memory_space=pltpu.VMEM),
            pl.BlockSpec(memory_space=pltpu.VMEM),
        ],
        out_specs=pl.BlockSpec(memory_space=pltpu.VMEM),
        scratch_shapes=[
            pltpu.VMEM((N_DEV - 1, n, hc), jnp.bfloat16),
            pltpu.VMEM((N_DEV - 1, n, hc), jnp.bfloat16),
            pltpu.VMEM((n, hc), jnp.bfloat16),
            pltpu.VMEM((N_DEV - 1, n, hc), jnp.bfloat16),
            pltpu.SemaphoreType.DMA((N_DEV - 1, S)),
            pltpu.SemaphoreType.DMA((N_DEV - 1, S)),
            pltpu.SemaphoreType.DMA((N_DEV - 1, S)),
            pltpu.SemaphoreType.DMA((N_DEV - 1, S)),
        ],
        compiler_params=pltpu.CompilerParams(collective_id=0),
    )(x, route_idx, expert_W)


# device time: 15722 ns/iter; 1.0349x vs baseline; 1.0349x over previous
import jax
import jax.numpy as jnp
from jax import lax
from jax.experimental import pallas as pl
from jax.experimental.pallas import tpu as pltpu

N_DEV = 4
S = 2


def kernel(x, router_W, route_idx, expert_W):
    del router_W
    n, d = x.shape
    e_per, _, h = expert_W.shape
    hc = h // N_DEV
    nr = n // S

    def body(x_ref, idx_ref, w_ref, out_ref, stage_ref, rs_comm_ref,
             ag_stage_ref, ag_comm_ref, rs_send_sems, rs_recv_sems,
             ag_send_sems, ag_recv_sems):
        my_pos = lax.axis_index("i")

        barrier_sem = pltpu.get_barrier_semaphore()
        for o in range(1, N_DEV):
            pl.semaphore_signal(
                barrier_sem, inc=1,
                device_id=((my_pos + o) % N_DEV,),
                device_id_type=pl.DeviceIdType.MESH,
            )

        route = idx_ref[:, :]
        xv = x_ref[:, :]
        xm = [
            jnp.where(route == my_pos * e_per + e, xv, 0.0).astype(jnp.bfloat16)
            for e in range(e_per)
        ]

        def partial_chunk(col_start):
            chunk = jnp.zeros((n, hc), dtype=jnp.float32)
            for e in range(e_per):
                chunk = chunk + jnp.dot(
                    xm[e], w_ref[e, :, pl.ds(col_start, hc)].astype(jnp.bfloat16),
                    preferred_element_type=jnp.float32,
                )
            return chunk

        for c in range(N_DEV):
            stage_ref[c, :, :] = partial_chunk(c * hc).astype(jnp.bfloat16)

        pl.semaphore_wait(barrier_sem, N_DEV - 1)

        for c in range(N_DEV):
            o = (c - my_pos) % N_DEV

            @pl.when(o != 0)
            def _(c=c, o=o):
                for s in range(S):
                    rdma = pltpu.make_async_remote_copy(
                        src_ref=stage_ref.at[c, pl.ds(s * nr, nr), :],
                        dst_ref=rs_comm_ref.at[o - 1, pl.ds(s * nr, nr), :],
                        send_sem=rs_send_sems.at[o - 1, s],
                        recv_sem=rs_recv_sems.at[o - 1, s],
                        device_id=(c,),
                        device_id_type=pl.DeviceIdType.MESH,
                    )
                    rdma.start()

        own = stage_ref[my_pos].astype(jnp.float32)

        for s in range(S):
            rows = pl.ds(s * nr, nr)
            for o in range(1, N_DEV):
                recv = pltpu.make_async_remote_copy(
                    src_ref=stage_ref.at[0, rows, :],
                    dst_ref=rs_comm_ref.at[o - 1, rows, :],
                    send_sem=rs_send_sems.at[o - 1, s],
                    recv_sem=rs_recv_sems.at[o - 1, s],
                    device_id=((my_pos + o) % N_DEV,),
                    device_id_type=pl.DeviceIdType.MESH,
                )
                recv.wait_recv()
            final = own[s * nr:(s + 1) * nr, :]
            for o in range(1, N_DEV):
                final = final + rs_comm_ref[o - 1, rows, :].astype(jnp.float32)
            finalb = final.astype(jnp.bfloat16)
            ag_stage_ref[rows, :] = finalb
            for o in range(1, N_DEV):
                rdma = pltpu.make_async_remote_copy(
                    src_ref=ag_stage_ref.at[rows, :],
                    dst_ref=ag_comm_ref.at[o - 1, rows, :],
                    send_sem=ag_send_sems.at[o - 1, s],
                    recv_sem=ag_recv_sems.at[o - 1, s],
                    device_id=((my_pos + o) % N_DEV,),
                    device_id_type=pl.DeviceIdType.MESH,
                )
                rdma.start()
            out_ref[rows, pl.ds(my_pos * hc, hc)] = finalb

        for s in range(S):
            rows = pl.ds(s * nr, nr)
            for o in range(1, N_DEV):
                recv = pltpu.make_async_remote_copy(
                    src_ref=ag_stage_ref.at[rows, :],
                    dst_ref=ag_comm_ref.at[o - 1, rows, :],
                    send_sem=ag_send_sems.at[o - 1, s],
                    recv_sem=ag_recv_sems.at[o - 1, s],
                    device_id=((my_pos + o) % N_DEV,),
                    device_id_type=pl.DeviceIdType.MESH,
                )
                recv.wait_recv()
                p = (my_pos - o) % N_DEV
                out_ref[rows, pl.ds(p * hc, hc)] = ag_comm_ref[o - 1, rows, :]

        for k in range(N_DEV - 1):
            for s in range(S):
                for sems in (rs_send_sems, ag_send_sems):
                    drain = pltpu.make_async_remote_copy(
                        src_ref=ag_stage_ref.at[pl.ds(s * nr, nr), :],
                        dst_ref=ag_comm_ref.at[k, pl.ds(s * nr, nr), :],
                        send_sem=sems.at[k, s],
                        recv_sem=ag_recv_sems.at[k, s],
                        device_id=(0,),
                        device_id_type=pl.DeviceIdType.MESH,
                    )
                    drain.wait_send()

    return pl.pallas_call(
        body,
        out_shape=jax.ShapeDtypeStruct((n, h), jnp.bfloat16),
        in_specs=[
            pl.BlockSpec(memory_space=pltpu.VMEM),
            pl.BlockSpec(memory_space=pltpu.VMEM),
            pl.BlockSpec(memory_space=pltpu.VMEM),
        ],
        out_specs=pl.BlockSpec(memory_space=pltpu.VMEM),
        scratch_shapes=[
            pltpu.VMEM((N_DEV, n, hc), jnp.bfloat16),
            pltpu.VMEM((N_DEV - 1, n, hc), jnp.bfloat16),
            pltpu.VMEM((n, hc), jnp.bfloat16),
            pltpu.VMEM((N_DEV - 1, n, hc), jnp.bfloat16),
            pltpu.SemaphoreType.DMA((N_DEV - 1, S)),
            pltpu.SemaphoreType.DMA((N_DEV - 1, S)),
            pltpu.SemaphoreType.DMA((N_DEV - 1, S)),
            pltpu.SemaphoreType.DMA((N_DEV - 1, S)),
        ],
        compiler_params=pltpu.CompilerParams(collective_id=0),
    )(x, route_idx, expert_W)
